# baseline (speedup 1.0000x reference)
import functools

import jax
import jax.numpy as jnp
from jax.experimental import pallas as pl
from jax.experimental.pallas import tpu as pltpu

_HI = jax.lax.Precision.HIGHEST
_INTERPRET = False

BLK1 = 256
BLK2 = 512


def _mm(a, b):
    return jax.lax.dot_general(a, b, (((1,), (0,)), ((), ())),
                               precision=_HI, preferred_element_type=jnp.float32)


def _mmT(a, b):
    return jax.lax.dot_general(a, b, (((1,), (1,)), ((), ())),
                               precision=_HI, preferred_element_type=jnp.float32)


def _mm_bf(a, b):
    return jax.lax.dot_general(a.astype(jnp.bfloat16), b.astype(jnp.bfloat16),
                               (((1,), (0,)), ((), ())),
                               preferred_element_type=jnp.float32)


def _mmT_bf(a, b):
    return jax.lax.dot_general(a.astype(jnp.bfloat16), b.astype(jnp.bfloat16),
                               (((1,), (1,)), ((), ())),
                               preferred_element_type=jnp.float32)


def _k1_body(locc_ref, loct_ref, feat_ref, p0_ref, p1_ref,
             w0a_ref, w0b_ref, w0c0_ref, w0c1_ref, b0_ref,
             w1_ref, b1_ref, w2_ref, b2_ref, w3_ref, b3_ref, w4_ref, b4_ref,
             out_ref):
    gx = locc_ref[0, 0, :].astype(jnp.bfloat16).astype(jnp.float32)
    gy = locc_ref[0, 1, :].astype(jnp.bfloat16).astype(jnp.float32)

    def pool_sample(pref, hw_side, hw_flat):
        side = float(hw_side)
        ix = jnp.clip(((gx + 1.0) * side - 1.0) / 2.0, 0.0, side - 1.0)
        iy = jnp.clip(((gy + 1.0) * side - 1.0) / 2.0, 0.0, side - 1.0)
        ix0f = jnp.floor(ix)
        iy0f = jnp.floor(iy)
        wx = ix - ix0f
        wy = iy - iy0f
        x0 = ix0f.astype(jnp.int32)
        x1 = jnp.minimum(ix0f + 1.0, side - 1.0).astype(jnp.int32)
        y0 = iy0f.astype(jnp.int32)
        y1 = jnp.minimum(iy0f + 1.0, side - 1.0).astype(jnp.int32)
        cio = jax.lax.broadcasted_iota(jnp.int32, (BLK1, hw_flat), 1)
        f00 = (y0 * hw_side + x0)[:, None]
        f01 = (y0 * hw_side + x1)[:, None]
        f10 = (y1 * hw_side + x0)[:, None]
        f11 = (y1 * hw_side + x1)[:, None]
        P = (jnp.where(cio == f00, ((1.0 - wx) * (1.0 - wy))[:, None], 0.0)
             + jnp.where(cio == f01, (wx * (1.0 - wy))[:, None], 0.0)
             + jnp.where(cio == f10, ((1.0 - wx) * wy)[:, None], 0.0)
             + jnp.where(cio == f11, (wx * wy)[:, None], 0.0))
        return _mmT(P, pref[0])

    pooled0 = pool_sample(p0_ref, 64, 4096)
    pooled1 = pool_sample(p1_ref, 32, 1024)
    loct = loct_ref[0]
    feat = feat_ref[0].T
    h = (_mm_bf(loct, w0a_ref[...]) + _mm_bf(feat, w0b_ref[...])
         + _mm_bf(pooled0, w0c0_ref[...]) + _mm_bf(pooled1, w0c1_ref[...]) + b0_ref[...])
    h = jnp.maximum(h, 0.0)
    h = jnp.maximum(_mm_bf(h, w1_ref[...]) + b1_ref[...], 0.0)
    h = jnp.maximum(_mm_bf(h, w2_ref[...]) + b2_ref[...], 0.0)
    h = jnp.maximum(_mm_bf(h, w3_ref[...]) + b3_ref[...], 0.0)
    h = _mm_bf(h, w4_ref[...]) + b4_ref[...]
    out_ref[0] = h


def _gc_body(loca_ref, xfa_ref, locb_ref, xfb_ref,
             rwl_ref, rwf_ref, rb_ref, twl_ref, twf_ref,
             *rest, n, blk, last):
    if last:
        lwl_ref, lwf_ref, lb_ref, out_ref, loc_out_ref, xhi_s, xlo_s, xlo2_s = rest
    else:
        out_ref, xhi_s, xlo_s, xlo2_s = rest
    loca = loca_ref[0]
    xfa = xfa_ref[0]
    locb = locb_ref[0]
    xfb = xfb_ref[0]
    nblk = pl.program_id(1)

    @pl.when(nblk == 0)
    def _decompose():
        hi = xfa.astype(jnp.bfloat16)
        r1 = xfa - hi.astype(jnp.float32)
        lo = r1.astype(jnp.bfloat16)
        lo2 = (r1 - lo.astype(jnp.float32)).astype(jnp.bfloat16)
        xhi_s[...] = hi
        xlo_s[...] = lo
        xlo2_s[...] = lo2

    sqa = jnp.sum(loca * loca, axis=1) + jnp.sum(xfa * xfa, axis=1)
    sqb = jnp.sum(locb * locb, axis=1) + jnp.sum(xfb * xfb, axis=1)
    d = sqb[:, None] + sqa[None, :] - 2.0 * (_mmT_bf(locb, loca) + _mmT_bf(xfb, xhi_s[...]))
    rowg = nblk * blk + jax.lax.broadcasted_iota(jnp.int32, (blk, 1), 0)
    cio = jax.lax.broadcasted_iota(jnp.int32, (blk, n), 1)
    d = d + jnp.where(cio == rowg, 1e10, 0.0)
    A = jnp.zeros((blk, n), jnp.float32)
    for _ in range(3):
        m = jnp.min(d, axis=1)
        amin = jnp.min(jnp.where(d <= m[:, None], cio, n), axis=1)
        e = cio == amin[:, None]
        A = A + e.astype(jnp.float32)
        d = jnp.where(e, 1e30, d)
    Abf = A.astype(jnp.bfloat16)

    def _agg(x_ref):
        return jax.lax.dot_general(Abf, x_ref[...], (((1,), (0,)), ((), ())),
                                   preferred_element_type=jnp.float32)

    aggr_xf = _agg(xhi_s) + _agg(xlo_s) + _agg(xlo2_s)
    out = (_mm_bf(_mm(A, loca), rwl_ref[...]) + _mm_bf(aggr_xf, rwf_ref[...])
           + rb_ref[...] + _mm_bf(locb, twl_ref[...]) + _mm_bf(xfb, twf_ref[...]))
    out = jnp.maximum(out, 0.0)
    out_ref[0] = out
    if last:
        nl = _mm_bf(locb, lwl_ref[...]) + _mm_bf(out, lwf_ref[...]) + lb_ref[...]
        loc_out_ref[0] = locb + jnp.tanh(nl)


def _full_spec(n, c):
    return pl.BlockSpec((1, n, c), lambda b, nb: (b, 0, 0))


def _blk_spec(blk, c):
    return pl.BlockSpec((1, blk, c), lambda b, nb: (b, nb, 0))


def _w_spec(shape):
    return pl.BlockSpec(shape, lambda b, nb: tuple(0 for _ in shape))


def kernel(pcd_location, pcd_features, x_to_pool_0, x_to_pool_1, params):
    p = params
    B, _, N = pcd_location.shape
    IN_CH = pcd_features.shape[1]
    loc_t = pcd_location.transpose(0, 2, 1)
    p0f = x_to_pool_0.reshape(B, 128, 64 * 64)
    p1f = x_to_pool_1.reshape(B, 128, 32 * 32)

    w0t = p['lin_w0'].T
    w0a, w0b = w0t[:3], w0t[3:3 + IN_CH]
    w0c0, w0c1 = w0t[3 + IN_CH:3 + IN_CH + 128], w0t[3 + IN_CH + 128:]
    mlp_ws = [w0a, w0b, w0c0, w0c1, p['lin_b0'][None, :]]
    for i in range(1, 5):
        mlp_ws += [p['lin_w%d' % i].T, p['lin_b%d' % i][None, :]]

    nf = pl.pallas_call(
        _k1_body,
        grid=(B, N // BLK1),
        in_specs=[
            pl.BlockSpec((1, 3, BLK1), lambda b, nb: (b, 0, nb)),
            _blk_spec(BLK1, 3),
            pl.BlockSpec((1, IN_CH, BLK1), lambda b, nb: (b, 0, nb)),
            _full_spec(128, 64 * 64),
            _full_spec(128, 32 * 32),
        ] + [_w_spec(w.shape) for w in mlp_ws],
        out_specs=_blk_spec(BLK1, 256),
        out_shape=jax.ShapeDtypeStruct((B, N, 256), jnp.float32),
        interpret=_INTERPRET,
    )(pcd_location, loc_t, pcd_features, p0f, p1f, *mlp_ws)

    xf = nf
    loc_new_t = None
    f3 = None
    for r in range(3):
        C = xf.shape[2]
        OC = p['gc%d_rel_w' % r].shape[0]
        last = (r == 2)
        relT = p['gc%d_rel_w' % r].T
        rootT = p['gc%d_root_w' % r].T
        ws = [relT[:3], relT[3:], p['gc%d_rel_b' % r][None, :], rootT[:3], rootT[3:]]
        out_shapes = [jax.ShapeDtypeStruct((B, N, OC), jnp.float32)]
        out_specs = [_blk_spec(BLK2, OC)]
        if last:
            locT = p['loc_w'].T
            ws += [locT[:3], locT[3:], p['loc_b'][None, :]]
            out_shapes.append(jax.ShapeDtypeStruct((B, N, 3), jnp.float32))
            out_specs.append(_blk_spec(BLK2, 3))
        body = functools.partial(_gc_body, n=N, blk=BLK2, last=last)
        res = pl.pallas_call(
            body,
            grid=(B, N // BLK2),
            in_specs=[
                _full_spec(N, 3),
                _full_spec(N, C),
                _blk_spec(BLK2, 3),
                _blk_spec(BLK2, C),
            ] + [_w_spec(w.shape) for w in ws],
            out_specs=out_specs,
            out_shape=out_shapes,
            scratch_shapes=[pltpu.VMEM((N, C), jnp.bfloat16)] * 3,
            interpret=_INTERPRET,
        )(loc_t, xf, loc_t, xf, *ws)
        if last:
            f3, loc_new_t = res
        else:
            xf = res[0]

    return loc_new_t.transpose(0, 2, 1), f3.transpose(0, 2, 1)


# speedup vs baseline: 1.0203x; 1.0203x over previous
import functools

import jax
import jax.numpy as jnp
from jax.experimental import pallas as pl
from jax.experimental.pallas import tpu as pltpu

_HI = jax.lax.Precision.HIGHEST
_INTERPRET = False

BLK1 = 512
BLK2 = 1024


def _mm(a, b):
    return jax.lax.dot_general(a, b, (((1,), (0,)), ((), ())),
                               precision=_HI, preferred_element_type=jnp.float32)


def _mmT(a, b):
    return jax.lax.dot_general(a, b, (((1,), (1,)), ((), ())),
                               precision=_HI, preferred_element_type=jnp.float32)


def _mm_bf(a, b):
    return jax.lax.dot_general(a.astype(jnp.bfloat16), b.astype(jnp.bfloat16),
                               (((1,), (0,)), ((), ())),
                               preferred_element_type=jnp.float32)


def _mmT_bf(a, b):
    return jax.lax.dot_general(a.astype(jnp.bfloat16), b.astype(jnp.bfloat16),
                               (((1,), (1,)), ((), ())),
                               preferred_element_type=jnp.float32)


def _k1_body(locc_ref, loct_ref, feat_ref, p0_ref, p1_ref,
             w0a_ref, w0b_ref, w0c0_ref, w0c1_ref, b0_ref,
             w1_ref, b1_ref, w2_ref, b2_ref, w3_ref, b3_ref, w4_ref, b4_ref,
             out_ref):
    gx = locc_ref[0, 0, :].astype(jnp.bfloat16).astype(jnp.float32)
    gy = locc_ref[0, 1, :].astype(jnp.bfloat16).astype(jnp.float32)

    def pool_sample(pref, hw_side, hw_flat):
        side = float(hw_side)
        ix = jnp.clip(((gx + 1.0) * side - 1.0) / 2.0, 0.0, side - 1.0)
        iy = jnp.clip(((gy + 1.0) * side - 1.0) / 2.0, 0.0, side - 1.0)
        ix0f = jnp.floor(ix)
        iy0f = jnp.floor(iy)
        wx = ix - ix0f
        wy = iy - iy0f
        x0 = ix0f.astype(jnp.int32)
        x1 = jnp.minimum(ix0f + 1.0, side - 1.0).astype(jnp.int32)
        y0 = iy0f.astype(jnp.int32)
        y1 = jnp.minimum(iy0f + 1.0, side - 1.0).astype(jnp.int32)
        cio = jax.lax.broadcasted_iota(jnp.int32, (BLK1, hw_flat), 1)
        f00 = (y0 * hw_side + x0)[:, None]
        f01 = (y0 * hw_side + x1)[:, None]
        f10 = (y1 * hw_side + x0)[:, None]
        f11 = (y1 * hw_side + x1)[:, None]
        P = (jnp.where(cio == f00, ((1.0 - wx) * (1.0 - wy))[:, None], 0.0)
             + jnp.where(cio == f01, (wx * (1.0 - wy))[:, None], 0.0)
             + jnp.where(cio == f10, ((1.0 - wx) * wy)[:, None], 0.0)
             + jnp.where(cio == f11, (wx * wy)[:, None], 0.0))
        return _mmT(P, pref[0])

    pooled0 = pool_sample(p0_ref, 64, 4096)
    pooled1 = pool_sample(p1_ref, 32, 1024)
    loct = loct_ref[0]
    feat = feat_ref[0].T
    h = (_mm_bf(loct, w0a_ref[...]) + _mm_bf(feat, w0b_ref[...])
         + _mm_bf(pooled0, w0c0_ref[...]) + _mm_bf(pooled1, w0c1_ref[...]) + b0_ref[...])
    h = jnp.maximum(h, 0.0)
    h = jnp.maximum(_mm_bf(h, w1_ref[...]) + b1_ref[...], 0.0)
    h = jnp.maximum(_mm_bf(h, w2_ref[...]) + b2_ref[...], 0.0)
    h = jnp.maximum(_mm_bf(h, w3_ref[...]) + b3_ref[...], 0.0)
    h = _mm_bf(h, w4_ref[...]) + b4_ref[...]
    out_ref[0] = h


def _gc_body(loca_ref, xfa_ref, locb_ref, xfb_ref,
             rwl_ref, rwf_ref, rb_ref, twl_ref, twf_ref,
             *rest, n, blk, last):
    if last:
        lwl_ref, lwf_ref, lb_ref, out_ref, loc_out_ref = rest
    else:
        (out_ref,) = rest
    loca = loca_ref[0]
    xfa = xfa_ref[0]
    locb = locb_ref[0]
    xfb = xfb_ref[0]
    nblk = pl.program_id(1)
    sqa = jnp.sum(loca * loca, axis=1) + jnp.sum(xfa * xfa, axis=1)
    sqb = jnp.sum(locb * locb, axis=1) + jnp.sum(xfb * xfb, axis=1)
    d = sqb[:, None] + sqa[None, :] - 2.0 * (_mmT_bf(locb, loca) + _mmT_bf(xfb, xfa))
    rowg = nblk * blk + jax.lax.broadcasted_iota(jnp.int32, (blk, 1), 0)
    cio = jax.lax.broadcasted_iota(jnp.int32, (blk, n), 1)
    d = d + jnp.where(cio == rowg, 1e10, 0.0)
    A = jnp.zeros((blk, n), jnp.float32)
    for _ in range(3):
        m = jnp.min(d, axis=1)
        amin = jnp.min(jnp.where(d <= m[:, None], cio, n), axis=1)
        e = cio == amin[:, None]
        A = A + e.astype(jnp.float32)
        d = jnp.where(e, 1e30, d)
    out = (_mm_bf(_mm(A, loca), rwl_ref[...]) + _mm_bf(_mm(A, xfa), rwf_ref[...])
           + rb_ref[...] + _mm_bf(locb, twl_ref[...]) + _mm_bf(xfb, twf_ref[...]))
    out = jnp.maximum(out, 0.0)
    out_ref[0] = out
    if last:
        nl = _mm_bf(locb, lwl_ref[...]) + _mm_bf(out, lwf_ref[...]) + lb_ref[...]
        loc_out_ref[0] = locb + jnp.tanh(nl)


def _full_spec(n, c):
    return pl.BlockSpec((1, n, c), lambda b, nb: (b, 0, 0))


def _blk_spec(blk, c):
    return pl.BlockSpec((1, blk, c), lambda b, nb: (b, nb, 0))


def _w_spec(shape):
    return pl.BlockSpec(shape, lambda b, nb: tuple(0 for _ in shape))


def kernel(pcd_location, pcd_features, x_to_pool_0, x_to_pool_1, params):
    p = params
    B, _, N = pcd_location.shape
    IN_CH = pcd_features.shape[1]
    loc_t = pcd_location.transpose(0, 2, 1)
    p0f = x_to_pool_0.reshape(B, 128, 64 * 64)
    p1f = x_to_pool_1.reshape(B, 128, 32 * 32)

    w0t = p['lin_w0'].T
    w0a, w0b = w0t[:3], w0t[3:3 + IN_CH]
    w0c0, w0c1 = w0t[3 + IN_CH:3 + IN_CH + 128], w0t[3 + IN_CH + 128:]
    mlp_ws = [w0a, w0b, w0c0, w0c1, p['lin_b0'][None, :]]
    for i in range(1, 5):
        mlp_ws += [p['lin_w%d' % i].T, p['lin_b%d' % i][None, :]]

    nf = pl.pallas_call(
        _k1_body,
        grid=(B, N // BLK1),
        in_specs=[
            pl.BlockSpec((1, 3, BLK1), lambda b, nb: (b, 0, nb)),
            _blk_spec(BLK1, 3),
            pl.BlockSpec((1, IN_CH, BLK1), lambda b, nb: (b, 0, nb)),
            _full_spec(128, 64 * 64),
            _full_spec(128, 32 * 32),
        ] + [_w_spec(w.shape) for w in mlp_ws],
        out_specs=_blk_spec(BLK1, 256),
        out_shape=jax.ShapeDtypeStruct((B, N, 256), jnp.float32),
        interpret=_INTERPRET,
    )(pcd_location, loc_t, pcd_features, p0f, p1f, *mlp_ws)

    xf = nf
    loc_new_t = None
    f3 = None
    for r in range(3):
        C = xf.shape[2]
        OC = p['gc%d_rel_w' % r].shape[0]
        last = (r == 2)
        relT = p['gc%d_rel_w' % r].T
        rootT = p['gc%d_root_w' % r].T
        ws = [relT[:3], relT[3:], p['gc%d_rel_b' % r][None, :], rootT[:3], rootT[3:]]
        out_shapes = [jax.ShapeDtypeStruct((B, N, OC), jnp.float32)]
        out_specs = [_blk_spec(BLK2, OC)]
        if last:
            locT = p['loc_w'].T
            ws += [locT[:3], locT[3:], p['loc_b'][None, :]]
            out_shapes.append(jax.ShapeDtypeStruct((B, N, 3), jnp.float32))
            out_specs.append(_blk_spec(BLK2, 3))
        body = functools.partial(_gc_body, n=N, blk=BLK2, last=last)
        res = pl.pallas_call(
            body,
            grid=(B, N // BLK2),
            in_specs=[
                _full_spec(N, 3),
                _full_spec(N, C),
                _blk_spec(BLK2, 3),
                _blk_spec(BLK2, C),
            ] + [_w_spec(w.shape) for w in ws],
            out_specs=out_specs,
            out_shape=out_shapes,
            interpret=_INTERPRET,
        )(loc_t, xf, loc_t, xf, *ws)
        if last:
            f3, loc_new_t = res
        else:
            xf = res[0]

    return loc_new_t.transpose(0, 2, 1), f3.transpose(0, 2, 1)
